# baseline (speedup 1.0000x reference)
import functools

import jax
import jax.numpy as jnp
from jax import lax
from jax.experimental import pallas as pl
from jax.experimental.pallas import tpu as pltpu
from jax.experimental.pallas import tpu_sc as plsc

NC = 2
NS = 16
NW = NC * NS

N_SLICES = 5


def _sc_gather_t(xt, table, batch, hs, embed_dim):
    bw = batch // NW
    mesh = plsc.VectorSubcoreMesh(core_axis_name="c", subcore_axis_name="s")

    @functools.partial(
        pl.kernel,
        mesh=mesh,
        out_type=jax.ShapeDtypeStruct((hs, embed_dim, batch), jnp.float32),
        scratch_types=[
            pltpu.VMEM((bw,), jnp.int32),
            pltpu.VMEM((bw, embed_dim), jnp.float32),
            pltpu.VMEM((embed_dim, bw), jnp.float32),
            pltpu.SemaphoreType.DMA,
        ],
        compiler_params=pltpu.CompilerParams(
            use_tc_tiling_on_sc=False, needs_layout_passes=False
        ),
    )
    def k(table_hbm, xt_hbm, out_hbm, idx_v, rows_v, stage_v, sem):
        wid = lax.axis_index("s") * NC + lax.axis_index("c")
        b0 = wid * bw
        iota = lax.iota(jnp.int32, 16)

        @pl.loop(0, hs)
        def _(h):
            pltpu.sync_copy(xt_hbm.at[h, pl.ds(b0, bw)], idx_v)
            pltpu.async_copy(table_hbm.at[idx_v], rows_v, sem).wait()

            @pl.loop(0, bw // 16)
            def _(kk):
                rvec = kk * 16 + iota
                for e in range(embed_dim):
                    vals = plsc.load_gather(
                        rows_v, [rvec, jnp.full((16,), e, jnp.int32)]
                    )
                    stage_v[e, pl.ds(kk * 16, 16)] = vals

            pltpu.sync_copy(stage_v, out_hbm.at[h, :, pl.ds(b0, bw)])

    return k(table, xt)


def kernel(x, table):
    batch, hist = x.shape
    vocab, embed_dim = table.shape
    hs = hist // N_SLICES
    xt = x.astype(jnp.int32).T
    outs = [
        _sc_gather_t(
            lax.slice(xt, (i * hs, 0), ((i + 1) * hs, batch)),
            table,
            batch,
            hs,
            embed_dim,
        )
        for i in range(N_SLICES)
    ]
    out_t = jnp.concatenate(outs, axis=0)
    return jnp.transpose(out_t, (2, 0, 1))


# speedup vs baseline: 1.7066x; 1.7066x over previous
import functools

import jax
import jax.numpy as jnp
from jax import lax
from jax.experimental import pallas as pl
from jax.experimental.pallas import tpu as pltpu
from jax.experimental.pallas import tpu_sc as plsc

NC = 2
NS = 16
NW = NC * NS

RB = 32


def _sc_gather(x, table, batch, hist, embed_dim):
    rows_per_w = batch // NW
    n_blocks = rows_per_w // RB
    assert n_blocks % 2 == 0 and n_blocks >= 4
    cp = RB * hist
    mesh = plsc.VectorSubcoreMesh(core_axis_name="c", subcore_axis_name="s")

    @functools.partial(
        pl.kernel,
        mesh=mesh,
        out_type=jax.ShapeDtypeStruct((batch, hist, embed_dim), jnp.float32),
        scratch_types=[
            *[pltpu.VMEM((cp,), jnp.int32) for _ in range(2)],
            *[pltpu.VMEM((cp, embed_dim), jnp.float32) for _ in range(2)],
            *[pltpu.SemaphoreType.DMA for _ in range(6)],
        ],
        compiler_params=pltpu.CompilerParams(use_tc_tiling_on_sc=False),
    )
    def k(table_hbm, x_hbm, out_hbm, i0, i1, r0, r1, si0, si1, sg0, sg1,
          so0, so1):
        idx_v = (i0, i1)
        rows_v = (r0, r1)
        sem_i = (si0, si1)
        sem_g = (sg0, sg1)
        sem_o = (so0, so1)
        wid = lax.axis_index("s") * NC + lax.axis_index("c")
        base = wid * rows_per_w

        def idx_start(t, b):
            b0 = base + t * RB
            for j in range(RB):
                pltpu.async_copy(
                    x_hbm.at[b0 + j],
                    idx_v[b].at[pl.ds(j * hist, hist)],
                    sem_i[b],
                )

        def idx_wait(b):
            for j in range(RB):
                pltpu.make_async_copy(
                    x_hbm.at[base],
                    idx_v[b].at[pl.ds(j * hist, hist)],
                    sem_i[b],
                ).wait()

        def gather_start(b):
            pltpu.async_copy(table_hbm.at[idx_v[b]], rows_v[b], sem_g[b])

        def gather_wait(b):
            pltpu.make_async_copy(
                table_hbm.at[idx_v[b]], rows_v[b], sem_g[b]
            ).wait()

        def out_start(t, b):
            b0 = base + t * RB
            for j in range(RB):
                pltpu.async_copy(
                    rows_v[b].at[pl.ds(j * hist, hist)],
                    out_hbm.at[b0 + j],
                    sem_o[b],
                )

        def out_wait(b):
            for j in range(RB):
                pltpu.make_async_copy(
                    rows_v[b].at[pl.ds(j * hist, hist)],
                    out_hbm.at[base],
                    sem_o[b],
                ).wait()

        idx_start(0, 0)
        idx_start(1, 1)
        idx_wait(0)
        gather_start(0)
        idx_wait(1)
        gather_start(1)
        gather_wait(0)
        out_start(0, 0)

        @pl.loop(1, n_blocks // 2)
        def _(p):
            t0 = 2 * p
            out_wait(0)
            idx_start(t0, 0)
            idx_wait(0)
            gather_start(0)
            gather_wait(1)
            out_start(t0 - 1, 1)
            idx_start(t0 + 1, 1)
            idx_wait(1)
            out_wait(1)
            gather_start(1)
            gather_wait(0)
            out_start(t0, 0)

        gather_wait(1)
        out_start(n_blocks - 1, 1)
        out_wait(0)
        out_wait(1)

    return k(table, x)


N_SLICES = 5


def kernel(x, table):
    batch, hist = x.shape
    vocab, embed_dim = table.shape
    xi = x.astype(jnp.int32)
    hs = hist // N_SLICES
    outs = [
        _sc_gather(
            lax.slice(xi, (0, i * hs), (batch, (i + 1) * hs)),
            table,
            batch,
            hs,
            embed_dim,
        )
        for i in range(N_SLICES)
    ]
    return jnp.concatenate(outs, axis=1)
